# Initial kernel scaffold; baseline (speedup 1.0000x reference)
import jax
import jax.numpy as jnp
from jax.experimental import pallas as pl


def kernel(x, edge_index, edge_weight, batch, W1_rel, b1, W1_root, W2_rel, b2, W2_root, Wg, bg, Wc1, bc1, Wc2, bc2):
    raise NotImplementedError("write your pallas kernel here")


# speedup vs baseline: 4.2161x; 4.2161x over previous
import functools

import jax
import jax.numpy as jnp
from jax import lax
from jax.experimental import pallas as pl
from jax.experimental.pallas import tpu as pltpu
from jax.experimental.pallas import tpu_sc as plsc

N = 10000
E = 320000
D = 128
G = 64

NC = 2
NS = 16
C = 80
EPT = E // (NC * NS)
NCHUNK = EPT // C
ROWS_PT = 624
ROWS_REM = N - NS * ROWS_PT

_sc_mesh = plsc.VectorSubcoreMesh(core_axis_name="c", subcore_axis_name="s")


@functools.partial(
    pl.kernel,
    out_type=jax.ShapeDtypeStruct((NC * N, D), jnp.float32),
    mesh=_sc_mesh,
    scratch_types=[
        pltpu.VMEM((C,), jnp.int32),
        pltpu.VMEM((C,), jnp.int32),
        pltpu.VMEM((C,), jnp.float32),
        pltpu.VMEM((C, D), jnp.float32),
        pltpu.VMEM_SHARED((N, D), jnp.float32),
        pltpu.SemaphoreType.DMA,
    ],
)
def _sc_edge_agg(x_hbm, src_hbm, dst_hbm, ew_hbm, out_hbm,
                 src_v, dst_v, ew_v, rows_v, acc_sh, sem):
    c = lax.axis_index("c")
    s = lax.axis_index("s")

    zeros16 = jnp.zeros((16,), jnp.float32)

    def zero_body(e, carry):
        for j in range(D // 16):
            rows_v[e, pl.ds(j * 16, 16)] = zeros16
        return carry

    lax.fori_loop(0, C, zero_body, 0)
    row0 = s * ROWS_PT
    for k in range(ROWS_PT // C):
        pltpu.sync_copy(rows_v, acc_sh.at[pl.ds(row0 + k * C, C)])
    rem = ROWS_PT % C
    if rem:
        pltpu.sync_copy(rows_v.at[pl.ds(0, rem)],
                        acc_sh.at[pl.ds(row0 + (ROWS_PT // C) * C, rem)])

    @pl.when(s == NS - 1)
    def _zero_tail():
        pltpu.sync_copy(rows_v.at[pl.ds(0, ROWS_REM)],
                        acc_sh.at[pl.ds(NS * ROWS_PT, ROWS_REM)])

    plsc.subcore_barrier()

    ebase = (c * NS + s) * EPT

    def chunk_body(i, carry):
        base = ebase + i * C
        pltpu.sync_copy(src_hbm.at[pl.ds(base, C)], src_v)
        pltpu.sync_copy(dst_hbm.at[pl.ds(base, C)], dst_v)
        pltpu.sync_copy(ew_hbm.at[pl.ds(base, C)], ew_v)
        pltpu.async_copy(x_hbm.at[src_v], rows_v, sem).wait()

        def scale_body(g, inner):
            ws = ew_v[pl.ds(g * 16, 16)]
            for ei in range(16):
                w = ws[ei]
                e = g * 16 + ei
                for j in range(D // 16):
                    rows_v[e, pl.ds(j * 16, 16)] = (
                        rows_v[e, pl.ds(j * 16, 16)] * w)
            return inner

        lax.fori_loop(0, C // 16, scale_body, 0)
        pltpu.sync_copy(rows_v, acc_sh.at[dst_v], add=True)
        return carry

    lax.fori_loop(0, NCHUNK, chunk_body, 0)
    plsc.subcore_barrier()

    out_base = c * N + s * ROWS_PT
    pltpu.sync_copy(acc_sh.at[pl.ds(s * ROWS_PT, ROWS_PT)],
                    out_hbm.at[pl.ds(out_base, ROWS_PT)])

    @pl.when(s == NS - 1)
    def _write_tail():
        pltpu.sync_copy(acc_sh.at[pl.ds(NS * ROWS_PT, ROWS_REM)],
                        out_hbm.at[pl.ds(c * N + NS * ROWS_PT, ROWS_REM)])


def _tc_layer_body(p_ref, x_ref, wr_ref, b_ref, ws_ref, o_ref):
    agg = p_ref[0:N, :] + p_ref[N:2 * N, :]
    acc = jnp.dot(agg, wr_ref[...], preferred_element_type=jnp.float32)
    acc = acc + jnp.dot(x_ref[...], ws_ref[...],
                        preferred_element_type=jnp.float32)
    o_ref[...] = jnp.maximum(acc + b_ref[...], 0.0)


_tc_layer = pl.pallas_call(
    _tc_layer_body,
    out_shape=jax.ShapeDtypeStruct((N, D), jnp.float32),
)


def _tc_head_body(p_ref, h1_ref, batch_ref, wr_ref, b2_ref, ws_ref,
                  wg_ref, bg_ref, wc1_ref, bc1_ref, wc2_ref, bc2_ref, o_ref):
    agg = p_ref[0:N, :] + p_ref[N:2 * N, :]
    h2 = jnp.dot(agg, wr_ref[...], preferred_element_type=jnp.float32)
    h2 = h2 + jnp.dot(h1_ref[...], ws_ref[...],
                      preferred_element_type=jnp.float32)
    h2 = jnp.maximum(h2 + b2_ref[...], 0.0)

    gate = jnp.dot(h2, wg_ref[...], preferred_element_type=jnp.float32)
    gate = gate + bg_ref[...]
    m = jnp.max(gate, axis=0, keepdims=True)
    e = jnp.exp(gate - m)

    seg_ids = lax.broadcasted_iota(jnp.int32, (N, G), 1)
    onehot = (batch_ref[...] == seg_ids).astype(jnp.float32)
    dims = (((0,), (0,)), ((), ()))
    den = lax.dot_general(onehot, e, dims,
                          preferred_element_type=jnp.float32)
    num = lax.dot_general(onehot, e * h2, dims,
                          preferred_element_type=jnp.float32)
    pooled = num / (den + 1e-16)

    z = jnp.maximum(jnp.dot(pooled, wc1_ref[...],
                            preferred_element_type=jnp.float32)
                    + bc1_ref[...], 0.0)
    logits = jnp.dot(z, wc2_ref[...],
                     preferred_element_type=jnp.float32) + bc2_ref[...]
    lmax = jnp.max(logits, axis=1, keepdims=True)
    le = jnp.exp(logits - lmax)
    o_ref[...] = le / jnp.sum(le, axis=1, keepdims=True)


_tc_head = pl.pallas_call(
    _tc_head_body,
    out_shape=jax.ShapeDtypeStruct((G, 2), jnp.float32),
)


def kernel(x, edge_index, edge_weight, batch,
           W1_rel, b1, W1_root, W2_rel, b2, W2_root,
           Wg, bg, Wc1, bc1, Wc2, bc2):
    src = edge_index[0].astype(jnp.int32)
    dst = edge_index[1].astype(jnp.int32)
    ew = edge_weight.astype(jnp.float32)
    batch2 = batch.astype(jnp.int32).reshape(N, 1)

    p1 = _sc_edge_agg(x, src, dst, ew)
    h1 = _tc_layer(p1, x, W1_rel, b1.reshape(1, D), W1_root)
    p2 = _sc_edge_agg(h1, src, dst, ew)
    out = _tc_head(p2, h1, batch2, W2_rel, b2.reshape(1, D), W2_root,
                   Wg, bg.reshape(1, D), Wc1, bc1.reshape(1, D),
                   Wc2, bc2.reshape(1, 2))
    return out


# speedup vs baseline: 6.3218x; 1.4995x over previous
import functools

import jax
import jax.numpy as jnp
from jax import lax
from jax.experimental import pallas as pl
from jax.experimental.pallas import tpu as pltpu
from jax.experimental.pallas import tpu_sc as plsc

N = 10000
E = 320000
D = 128
G = 64

NC = 2
NS = 16
NW = NC * NS
CH = 128
EPT = E // NW
NCHUNK = -(-EPT // CH)
EPT_PAD = NCHUNK * CH
ROWS_PT = 624
ROWS_REM = N - NS * ROWS_PT

_sc_mesh = plsc.VectorSubcoreMesh(core_axis_name="c", subcore_axis_name="s")


@functools.partial(
    pl.kernel,
    out_type=jax.ShapeDtypeStruct((NC * N, D), jnp.float32),
    mesh=_sc_mesh,
    scratch_types=[
        pltpu.VMEM((EPT_PAD,), jnp.int32),
        pltpu.VMEM((CH, D), jnp.float32),
        pltpu.VMEM((CH, D), jnp.float32),
        pltpu.VMEM((CH,), jnp.int32),
        pltpu.VMEM((CH,), jnp.int32),
        pltpu.VMEM((CH,), jnp.float32),
        pltpu.VMEM((CH,), jnp.float32),
        pltpu.VMEM_SHARED((N + 8, D), jnp.float32),
        pltpu.SemaphoreType.DMA,
        pltpu.SemaphoreType.DMA,
        pltpu.SemaphoreType.DMA,
        pltpu.SemaphoreType.DMA,
        pltpu.SemaphoreType.DMA,
        pltpu.SemaphoreType.DMA,
    ],
)
def _sc_edge_agg(x_hbm, src_hbm, dst_hbm, ew_hbm, out_hbm,
                 src_all, rows0, rows1, dstv0, dstv1, ewv0, ewv1, acc_sh,
                 gsem0, gsem1, dsem0, dsem1, esem0, esem1):
    c = lax.axis_index("c")
    s = lax.axis_index("s")
    tile = c * NS + s
    slab = tile * EPT_PAD

    pltpu.sync_copy(src_hbm.at[pl.ds(slab, EPT_PAD)], src_all)

    zeros16 = jnp.zeros((16,), jnp.float32)

    def zero_body(e, carry):
        for j in range(D // 16):
            rows1[e, pl.ds(j * 16, 16)] = zeros16
        return carry

    lax.fori_loop(0, CH, zero_body, 0)
    row0 = s * ROWS_PT
    for k in range(ROWS_PT // CH):
        pltpu.sync_copy(rows1, acc_sh.at[pl.ds(row0 + k * CH, CH)])
    rem = ROWS_PT % CH
    if rem:
        pltpu.sync_copy(rows1.at[pl.ds(0, rem)],
                        acc_sh.at[pl.ds(row0 + (ROWS_PT // CH) * CH, rem)])

    @pl.when(s == NS - 1)
    def _zero_tail():
        pltpu.sync_copy(rows1.at[pl.ds(0, ROWS_REM)],
                        acc_sh.at[pl.ds(NS * ROWS_PT, ROWS_REM)])

    def prefetch(chunk, dstv, ewv, dsem, esem):
        pltpu.async_copy(dst_hbm.at[pl.ds(slab + chunk * CH, CH)],
                         dstv, dsem)
        pltpu.async_copy(ew_hbm.at[pl.ds(slab + chunk * CH, CH)],
                         ewv, esem)

    def fire_gather(chunk, rows, sem):
        pltpu.async_copy(
            x_hbm.at[src_all.at[pl.ds(chunk * CH, CH)]], rows, sem)

    prefetch(0, dstv0, ewv0, dsem0, esem0)
    fire_gather(0, rows0, gsem0)
    plsc.subcore_barrier()

    def process(rows, dstv, ewv, gsem, dsem, esem):
        pltpu.make_async_copy(
            x_hbm.at[src_all.at[pl.ds(0, CH)]], rows, gsem).wait()
        pltpu.make_async_copy(
            dst_hbm.at[pl.ds(slab, CH)], dstv, dsem).wait()
        pltpu.make_async_copy(
            ew_hbm.at[pl.ds(slab, CH)], ewv, esem).wait()

        def scale_body(g, inner):
            ws = ewv[pl.ds(g * 16, 16)]
            for ei in range(16):
                w = ws[ei]
                e = g * 16 + ei
                for j in range(D // 16):
                    rows[e, pl.ds(j * 16, 16)] = (
                        rows[e, pl.ds(j * 16, 16)] * w)
            return inner

        lax.fori_loop(0, CH // 16, scale_body, 0)
        pltpu.sync_copy(rows, acc_sh.at[dstv], add=True)

    def pair_body(i2, carry):
        c0 = 2 * i2
        prefetch(c0 + 1, dstv1, ewv1, dsem1, esem1)
        fire_gather(c0 + 1, rows1, gsem1)
        process(rows0, dstv0, ewv0, gsem0, dsem0, esem0)
        prefetch(c0 + 2, dstv0, ewv0, dsem0, esem0)
        fire_gather(c0 + 2, rows0, gsem0)
        process(rows1, dstv1, ewv1, gsem1, dsem1, esem1)
        return carry

    lax.fori_loop(0, (NCHUNK - 1) // 2, pair_body, 0)
    process(rows0, dstv0, ewv0, gsem0, dsem0, esem0)
    plsc.subcore_barrier()

    out_base = c * N + s * ROWS_PT
    pltpu.sync_copy(acc_sh.at[pl.ds(s * ROWS_PT, ROWS_PT)],
                    out_hbm.at[pl.ds(out_base, ROWS_PT)])

    @pl.when(s == NS - 1)
    def _write_tail():
        pltpu.sync_copy(acc_sh.at[pl.ds(NS * ROWS_PT, ROWS_REM)],
                        out_hbm.at[pl.ds(c * N + NS * ROWS_PT, ROWS_REM)])


def _tc_layer_body(p_ref, x_ref, wr_ref, b_ref, ws_ref, o_ref):
    agg = p_ref[0:N, :] + p_ref[N:2 * N, :]
    acc = jnp.dot(agg, wr_ref[...], preferred_element_type=jnp.float32)
    acc = acc + jnp.dot(x_ref[...], ws_ref[...],
                        preferred_element_type=jnp.float32)
    o_ref[...] = jnp.maximum(acc + b_ref[...], 0.0)


_tc_layer = pl.pallas_call(
    _tc_layer_body,
    out_shape=jax.ShapeDtypeStruct((N, D), jnp.float32),
)


def _tc_head_body(p_ref, h1_ref, batch_ref, wr_ref, b2_ref, ws_ref,
                  wg_ref, bg_ref, wc1_ref, bc1_ref, wc2_ref, bc2_ref, o_ref):
    agg = p_ref[0:N, :] + p_ref[N:2 * N, :]
    h2 = jnp.dot(agg, wr_ref[...], preferred_element_type=jnp.float32)
    h2 = h2 + jnp.dot(h1_ref[...], ws_ref[...],
                      preferred_element_type=jnp.float32)
    h2 = jnp.maximum(h2 + b2_ref[...], 0.0)

    gate = jnp.dot(h2, wg_ref[...], preferred_element_type=jnp.float32)
    gate = gate + bg_ref[...]
    m = jnp.max(gate, axis=0, keepdims=True)
    e = jnp.exp(gate - m)

    seg_ids = lax.broadcasted_iota(jnp.int32, (N, G), 1)
    onehot = (batch_ref[...] == seg_ids).astype(jnp.float32)
    dims = (((0,), (0,)), ((), ()))
    den = lax.dot_general(onehot, e, dims,
                          preferred_element_type=jnp.float32)
    num = lax.dot_general(onehot, e * h2, dims,
                          preferred_element_type=jnp.float32)
    pooled = num / (den + 1e-16)

    z = jnp.maximum(jnp.dot(pooled, wc1_ref[...],
                            preferred_element_type=jnp.float32)
                    + bc1_ref[...], 0.0)
    logits = jnp.dot(z, wc2_ref[...],
                     preferred_element_type=jnp.float32) + bc2_ref[...]
    lmax = jnp.max(logits, axis=1, keepdims=True)
    le = jnp.exp(logits - lmax)
    o_ref[...] = le / jnp.sum(le, axis=1, keepdims=True)


_tc_head = pl.pallas_call(
    _tc_head_body,
    out_shape=jax.ShapeDtypeStruct((G, 2), jnp.float32),
)


def kernel(x, edge_index, edge_weight, batch,
           W1_rel, b1, W1_root, W2_rel, b2, W2_root,
           Wg, bg, Wc1, bc1, Wc2, bc2):
    pad = EPT_PAD - EPT

    def _slab(a, fill):
        return jnp.pad(a.reshape(NW, EPT), ((0, 0), (0, pad)),
                       constant_values=fill).reshape(-1)

    src = _slab(edge_index[0].astype(jnp.int32), 0)
    dst = _slab(edge_index[1].astype(jnp.int32), N)
    ew = _slab(edge_weight.astype(jnp.float32), 0.0)
    batch2 = batch.astype(jnp.int32).reshape(N, 1)

    p1 = _sc_edge_agg(x, src, dst, ew)
    h1 = _tc_layer(p1, x, W1_rel, b1.reshape(1, D), W1_root)
    p2 = _sc_edge_agg(h1, src, dst, ew)
    out = _tc_head(p2, h1, batch2, W2_rel, b2.reshape(1, D), W2_root,
                   Wg, bg.reshape(1, D), Wc1, bc1.reshape(1, D),
                   Wc2, bc2.reshape(1, 2))
    return out


# speedup vs baseline: 7.5970x; 1.2017x over previous
import functools

import jax
import jax.numpy as jnp
from jax import lax
from jax.experimental import pallas as pl
from jax.experimental.pallas import tpu as pltpu
from jax.experimental.pallas import tpu_sc as plsc

N = 10000
E = 320000
D = 128
G = 64

NC = 2
NS = 16
NW = NC * NS
CH = 96
EPT = E // NW
NCHUNK = -(-EPT // CH)
EPT_PAD = NCHUNK * CH
ROWS_PT = 624
ROWS_REM = N - NS * ROWS_PT

_sc_mesh = plsc.VectorSubcoreMesh(core_axis_name="c", subcore_axis_name="s")


@functools.partial(
    pl.kernel,
    out_type=jax.ShapeDtypeStruct((NC * N, D), jnp.float32),
    mesh=_sc_mesh,
    scratch_types=[
        pltpu.VMEM((EPT_PAD,), jnp.int32),
        [pltpu.VMEM((CH, D), jnp.float32) for _ in range(3)],
        [pltpu.VMEM((CH,), jnp.int32) for _ in range(3)],
        [pltpu.VMEM((CH,), jnp.float32) for _ in range(3)],
        pltpu.VMEM_SHARED((N + 8, D), jnp.float32),
        [pltpu.SemaphoreType.DMA for _ in range(3)],
        [pltpu.SemaphoreType.DMA for _ in range(3)],
        [pltpu.SemaphoreType.DMA for _ in range(3)],
        [pltpu.SemaphoreType.DMA for _ in range(3)],
    ],
)
def _sc_edge_agg(x_hbm, src_hbm, dst_hbm, ew_hbm, out_hbm,
                 src_all, rows, dstv, ewv, acc_sh,
                 gsem, dsem, esem, ssem):
    c = lax.axis_index("c")
    s = lax.axis_index("s")
    tile = c * NS + s
    slab = tile * EPT_PAD

    pltpu.sync_copy(src_hbm.at[pl.ds(slab, EPT_PAD)], src_all)

    zeros16 = jnp.zeros((16,), jnp.float32)

    def zero_body(e, carry):
        for j in range(D // 16):
            rows[2][e, pl.ds(j * 16, 16)] = zeros16
        return carry

    lax.fori_loop(0, CH, zero_body, 0)
    row0 = s * ROWS_PT
    for k in range(ROWS_PT // CH):
        pltpu.sync_copy(rows[2], acc_sh.at[pl.ds(row0 + k * CH, CH)])
    rem = ROWS_PT % CH
    if rem:
        pltpu.sync_copy(rows[2].at[pl.ds(0, rem)],
                        acc_sh.at[pl.ds(row0 + (ROWS_PT // CH) * CH, rem)])

    @pl.when(s == NS - 1)
    def _zero_tail():
        pltpu.sync_copy(rows[2].at[pl.ds(0, ROWS_REM)],
                        acc_sh.at[pl.ds(NS * ROWS_PT, ROWS_REM)])

    def fire(chunk, b):
        pltpu.async_copy(dst_hbm.at[pl.ds(slab + chunk * CH, CH)],
                         dstv[b], dsem[b])
        pltpu.async_copy(ew_hbm.at[pl.ds(slab + chunk * CH, CH)],
                         ewv[b], esem[b])
        pltpu.async_copy(
            x_hbm.at[src_all.at[pl.ds(chunk * CH, CH)]], rows[b], gsem[b])

    def wait_scatter(b):
        pltpu.make_async_copy(rows[b], acc_sh.at[dstv[b]], ssem[b]).wait()

    fire(0, 0)
    fire(1, 1)
    plsc.subcore_barrier()

    def process(b):
        pltpu.make_async_copy(
            x_hbm.at[src_all.at[pl.ds(0, CH)]], rows[b], gsem[b]).wait()
        pltpu.make_async_copy(
            dst_hbm.at[pl.ds(slab, CH)], dstv[b], dsem[b]).wait()
        pltpu.make_async_copy(
            ew_hbm.at[pl.ds(slab, CH)], ewv[b], esem[b]).wait()

        def scale_body(g, inner):
            ws = ewv[b][pl.ds(g * 16, 16)]
            for ei in range(16):
                w = ws[ei]
                e = g * 16 + ei
                for j in range(D // 16):
                    rows[b][e, pl.ds(j * 16, 16)] = (
                        rows[b][e, pl.ds(j * 16, 16)] * w)
            return inner

        lax.fori_loop(0, CH // 16, scale_body, 0)
        pltpu.async_copy(rows[b], acc_sh.at[dstv[b]], ssem[b], add=True)

    def triple_body(i3, carry):
        i = 3 * i3
        process(0)

        @pl.when(i3 > 0)
        def _w0():
            wait_scatter(2)

        fire(i + 2, 2)
        process(1)
        wait_scatter(0)

        @pl.when(i3 < NCHUNK // 3 - 1)
        def _f1():
            fire(i + 3, 0)

        process(2)

        @pl.when(i3 < NCHUNK // 3 - 1)
        def _w2f2():
            wait_scatter(1)
            fire(i + 4, 1)

        return carry

    lax.fori_loop(0, NCHUNK // 3, triple_body, 0)
    wait_scatter(1)
    wait_scatter(2)
    plsc.subcore_barrier()

    out_base = c * N + s * ROWS_PT
    pltpu.sync_copy(acc_sh.at[pl.ds(s * ROWS_PT, ROWS_PT)],
                    out_hbm.at[pl.ds(out_base, ROWS_PT)])

    @pl.when(s == NS - 1)
    def _write_tail():
        pltpu.sync_copy(acc_sh.at[pl.ds(NS * ROWS_PT, ROWS_REM)],
                        out_hbm.at[pl.ds(c * N + NS * ROWS_PT, ROWS_REM)])


def _tc_layer_body(p_ref, x_ref, wr_ref, b_ref, ws_ref, o_ref):
    agg = p_ref[0:N, :] + p_ref[N:2 * N, :]
    acc = jnp.dot(agg, wr_ref[...], preferred_element_type=jnp.float32)
    acc = acc + jnp.dot(x_ref[...], ws_ref[...],
                        preferred_element_type=jnp.float32)
    o_ref[...] = jnp.maximum(acc + b_ref[...], 0.0)


_tc_layer = pl.pallas_call(
    _tc_layer_body,
    out_shape=jax.ShapeDtypeStruct((N, D), jnp.float32),
)


def _tc_head_body(p_ref, h1_ref, batch_ref, wr_ref, b2_ref, ws_ref,
                  wg_ref, bg_ref, wc1_ref, bc1_ref, wc2_ref, bc2_ref, o_ref):
    agg = p_ref[0:N, :] + p_ref[N:2 * N, :]
    h2 = jnp.dot(agg, wr_ref[...], preferred_element_type=jnp.float32)
    h2 = h2 + jnp.dot(h1_ref[...], ws_ref[...],
                      preferred_element_type=jnp.float32)
    h2 = jnp.maximum(h2 + b2_ref[...], 0.0)

    gate = jnp.dot(h2, wg_ref[...], preferred_element_type=jnp.float32)
    gate = gate + bg_ref[...]
    m = jnp.max(gate, axis=0, keepdims=True)
    e = jnp.exp(gate - m)

    seg_ids = lax.broadcasted_iota(jnp.int32, (N, G), 1)
    onehot = (batch_ref[...] == seg_ids).astype(jnp.float32)
    dims = (((0,), (0,)), ((), ()))
    den = lax.dot_general(onehot, e, dims,
                          preferred_element_type=jnp.float32)
    num = lax.dot_general(onehot, e * h2, dims,
                          preferred_element_type=jnp.float32)
    pooled = num / (den + 1e-16)

    z = jnp.maximum(jnp.dot(pooled, wc1_ref[...],
                            preferred_element_type=jnp.float32)
                    + bc1_ref[...], 0.0)
    logits = jnp.dot(z, wc2_ref[...],
                     preferred_element_type=jnp.float32) + bc2_ref[...]
    lmax = jnp.max(logits, axis=1, keepdims=True)
    le = jnp.exp(logits - lmax)
    o_ref[...] = le / jnp.sum(le, axis=1, keepdims=True)


_tc_head = pl.pallas_call(
    _tc_head_body,
    out_shape=jax.ShapeDtypeStruct((G, 2), jnp.float32),
)


def kernel(x, edge_index, edge_weight, batch,
           W1_rel, b1, W1_root, W2_rel, b2, W2_root,
           Wg, bg, Wc1, bc1, Wc2, bc2):
    pad = EPT_PAD - EPT

    def _slab(a, fill):
        return jnp.pad(a.reshape(NW, EPT), ((0, 0), (0, pad)),
                       constant_values=fill).reshape(-1)

    src = _slab(edge_index[0].astype(jnp.int32), 0)
    dst = _slab(edge_index[1].astype(jnp.int32), N)
    ew = _slab(edge_weight.astype(jnp.float32), 0.0)
    batch2 = batch.astype(jnp.int32).reshape(N, 1)

    p1 = _sc_edge_agg(x, src, dst, ew)
    h1 = _tc_layer(p1, x, W1_rel, b1.reshape(1, D), W1_root)
    p2 = _sc_edge_agg(h1, src, dst, ew)
    out = _tc_head(p2, h1, batch2, W2_rel, b2.reshape(1, D), W2_root,
                   Wg, bg.reshape(1, D), Wc1, bc1.reshape(1, D),
                   Wc2, bc2.reshape(1, 2))
    return out


# speedup vs baseline: 7.6157x; 1.0025x over previous
import functools

import jax
import jax.numpy as jnp
from jax import lax
from jax.experimental import pallas as pl
from jax.experimental.pallas import tpu as pltpu
from jax.experimental.pallas import tpu_sc as plsc

N = 10000
E = 320000
D = 128
G = 64

NC = 2
NS = 16
NW = NC * NS
CH = 96
EPT = E // NW
NCHUNK = -(-EPT // CH)
EPT_PAD = NCHUNK * CH
ROWS_PT = 624
ROWS_REM = N - NS * ROWS_PT

_sc_mesh = plsc.VectorSubcoreMesh(core_axis_name="c", subcore_axis_name="s")


@functools.partial(
    pl.kernel,
    out_type=jax.ShapeDtypeStruct((NC * N, D), jnp.float32),
    mesh=_sc_mesh,
    scratch_types=[
        pltpu.VMEM((EPT_PAD,), jnp.int32),
        [pltpu.VMEM((CH, D), jnp.float32) for _ in range(3)],
        [pltpu.VMEM((CH,), jnp.int32) for _ in range(3)],
        [pltpu.VMEM((CH,), jnp.float32) for _ in range(3)],
        pltpu.VMEM_SHARED((N + 8, D), jnp.float32),
        [pltpu.SemaphoreType.DMA for _ in range(3)],
        [pltpu.SemaphoreType.DMA for _ in range(3)],
        [pltpu.SemaphoreType.DMA for _ in range(3)],
        [pltpu.SemaphoreType.DMA for _ in range(3)],
        [pltpu.SemaphoreType.DMA for _ in range(3)],
    ],
)
def _sc_edge_agg(x_hbm, src_hbm, dst_hbm, ew_hbm, out_hbm,
                 src_all, rows, dstv, ewv, acc_sh,
                 gsem, gsem2, dsem, esem, ssem):
    c = lax.axis_index("c")
    s = lax.axis_index("s")
    tile = c * NS + s
    slab = tile * EPT_PAD

    pltpu.sync_copy(src_hbm.at[pl.ds(slab, EPT_PAD)], src_all)

    zeros16 = jnp.zeros((16,), jnp.float32)

    def zero_body(e, carry):
        for j in range(D // 16):
            rows[2][e, pl.ds(j * 16, 16)] = zeros16
        return carry

    lax.fori_loop(0, CH, zero_body, 0)
    row0 = s * ROWS_PT
    for k in range(ROWS_PT // CH):
        pltpu.sync_copy(rows[2], acc_sh.at[pl.ds(row0 + k * CH, CH)])
    rem = ROWS_PT % CH
    if rem:
        pltpu.sync_copy(rows[2].at[pl.ds(0, rem)],
                        acc_sh.at[pl.ds(row0 + (ROWS_PT // CH) * CH, rem)])

    @pl.when(s == NS - 1)
    def _zero_tail():
        pltpu.sync_copy(rows[2].at[pl.ds(0, ROWS_REM)],
                        acc_sh.at[pl.ds(NS * ROWS_PT, ROWS_REM)])

    def fire(chunk, b):
        pltpu.async_copy(dst_hbm.at[pl.ds(slab + chunk * CH, CH)],
                         dstv[b], dsem[b])
        pltpu.async_copy(ew_hbm.at[pl.ds(slab + chunk * CH, CH)],
                         ewv[b], esem[b])
        H = CH // 2
        pltpu.async_copy(
            x_hbm.at[src_all.at[pl.ds(chunk * CH, H)]],
            rows[b].at[pl.ds(0, H)], gsem[b])
        pltpu.async_copy(
            x_hbm.at[src_all.at[pl.ds(chunk * CH + H, H)]],
            rows[b].at[pl.ds(H, H)], gsem2[b])

    def wait_scatter(b):
        pltpu.make_async_copy(rows[b], acc_sh.at[dstv[b]], ssem[b]).wait()

    fire(0, 0)
    fire(1, 1)
    plsc.subcore_barrier()

    def process(b):
        H = CH // 2
        pltpu.make_async_copy(
            x_hbm.at[src_all.at[pl.ds(0, H)]],
            rows[b].at[pl.ds(0, H)], gsem[b]).wait()
        pltpu.make_async_copy(
            x_hbm.at[src_all.at[pl.ds(0, H)]],
            rows[b].at[pl.ds(H, H)], gsem2[b]).wait()
        pltpu.make_async_copy(
            dst_hbm.at[pl.ds(slab, CH)], dstv[b], dsem[b]).wait()
        pltpu.make_async_copy(
            ew_hbm.at[pl.ds(slab, CH)], ewv[b], esem[b]).wait()

        def scale_body(g, inner):
            ws = ewv[b][pl.ds(g * 16, 16)]
            for ei in range(16):
                w = ws[ei]
                e = g * 16 + ei
                for j in range(D // 16):
                    rows[b][e, pl.ds(j * 16, 16)] = (
                        rows[b][e, pl.ds(j * 16, 16)] * w)
            return inner

        lax.fori_loop(0, CH // 16, scale_body, 0)
        pltpu.async_copy(rows[b], acc_sh.at[dstv[b]], ssem[b], add=True)

    def triple_body(i3, carry):
        i = 3 * i3
        process(0)

        @pl.when(i3 > 0)
        def _w0():
            wait_scatter(2)

        fire(i + 2, 2)
        process(1)
        wait_scatter(0)

        @pl.when(i3 < NCHUNK // 3 - 1)
        def _f1():
            fire(i + 3, 0)

        process(2)

        @pl.when(i3 < NCHUNK // 3 - 1)
        def _w2f2():
            wait_scatter(1)
            fire(i + 4, 1)

        return carry

    lax.fori_loop(0, NCHUNK // 3, triple_body, 0)
    wait_scatter(1)
    wait_scatter(2)
    plsc.subcore_barrier()

    out_base = c * N + s * ROWS_PT
    pltpu.sync_copy(acc_sh.at[pl.ds(s * ROWS_PT, ROWS_PT)],
                    out_hbm.at[pl.ds(out_base, ROWS_PT)])

    @pl.when(s == NS - 1)
    def _write_tail():
        pltpu.sync_copy(acc_sh.at[pl.ds(NS * ROWS_PT, ROWS_REM)],
                        out_hbm.at[pl.ds(c * N + NS * ROWS_PT, ROWS_REM)])


def _tc_layer_body(p_ref, x_ref, wr_ref, b_ref, ws_ref, o_ref):
    agg = p_ref[0:N, :] + p_ref[N:2 * N, :]
    acc = jnp.dot(agg, wr_ref[...], preferred_element_type=jnp.float32)
    acc = acc + jnp.dot(x_ref[...], ws_ref[...],
                        preferred_element_type=jnp.float32)
    o_ref[...] = jnp.maximum(acc + b_ref[...], 0.0)


_tc_layer = pl.pallas_call(
    _tc_layer_body,
    out_shape=jax.ShapeDtypeStruct((N, D), jnp.float32),
)


def _tc_head_body(p_ref, h1_ref, batch_ref, wr_ref, b2_ref, ws_ref,
                  wg_ref, bg_ref, wc1_ref, bc1_ref, wc2_ref, bc2_ref, o_ref):
    agg = p_ref[0:N, :] + p_ref[N:2 * N, :]
    h2 = jnp.dot(agg, wr_ref[...], preferred_element_type=jnp.float32)
    h2 = h2 + jnp.dot(h1_ref[...], ws_ref[...],
                      preferred_element_type=jnp.float32)
    h2 = jnp.maximum(h2 + b2_ref[...], 0.0)

    gate = jnp.dot(h2, wg_ref[...], preferred_element_type=jnp.float32)
    gate = gate + bg_ref[...]
    m = jnp.max(gate, axis=0, keepdims=True)
    e = jnp.exp(gate - m)

    seg_ids = lax.broadcasted_iota(jnp.int32, (N, G), 1)
    onehot = (batch_ref[...] == seg_ids).astype(jnp.float32)
    dims = (((0,), (0,)), ((), ()))
    den = lax.dot_general(onehot, e, dims,
                          preferred_element_type=jnp.float32)
    num = lax.dot_general(onehot, e * h2, dims,
                          preferred_element_type=jnp.float32)
    pooled = num / (den + 1e-16)

    z = jnp.maximum(jnp.dot(pooled, wc1_ref[...],
                            preferred_element_type=jnp.float32)
                    + bc1_ref[...], 0.0)
    logits = jnp.dot(z, wc2_ref[...],
                     preferred_element_type=jnp.float32) + bc2_ref[...]
    lmax = jnp.max(logits, axis=1, keepdims=True)
    le = jnp.exp(logits - lmax)
    o_ref[...] = le / jnp.sum(le, axis=1, keepdims=True)


_tc_head = pl.pallas_call(
    _tc_head_body,
    out_shape=jax.ShapeDtypeStruct((G, 2), jnp.float32),
)


def kernel(x, edge_index, edge_weight, batch,
           W1_rel, b1, W1_root, W2_rel, b2, W2_root,
           Wg, bg, Wc1, bc1, Wc2, bc2):
    pad = EPT_PAD - EPT

    def _slab(a, fill):
        return jnp.pad(a.reshape(NW, EPT), ((0, 0), (0, pad)),
                       constant_values=fill).reshape(-1)

    src = _slab(edge_index[0].astype(jnp.int32), 0)
    dst = _slab(edge_index[1].astype(jnp.int32), N)
    ew = _slab(edge_weight.astype(jnp.float32), 0.0)
    batch2 = batch.astype(jnp.int32).reshape(N, 1)

    p1 = _sc_edge_agg(x, src, dst, ew)
    h1 = _tc_layer(p1, x, W1_rel, b1.reshape(1, D), W1_root)
    p2 = _sc_edge_agg(h1, src, dst, ew)
    out = _tc_head(p2, h1, batch2, W2_rel, b2.reshape(1, D), W2_root,
                   Wg, bg.reshape(1, D), Wc1, bc1.reshape(1, D),
                   Wc2, bc2.reshape(1, 2))
    return out
